# baseline (speedup 1.0000x reference)
import functools

import jax
import jax.numpy as jnp
from jax import lax
from jax.experimental import pallas as pl
from jax.experimental.pallas import tpu as pltpu
from jax.experimental.pallas import tpu_sc as plsc

VOCAB = 1000000
DIM = 64
BATCH = 16384

NC = 2
NS = 16
L = 16
NW = NC * NS

CW = 128
NCHUNK = VOCAB // CW
CPT = 245
TAIL_COL = NCHUNK * CW
TAIL_TILE = NW - 1

HCAP = BATCH + 16
ROWS_PAD = 128
NSLOT = BATCH + 8
DUMMY = BATCH
SCAP = 64
NOFF = 256


NBUF = 4


def _phase1_body(wt_hbm, ut_hbm, c_hbm, o_hbm, rowsA_hbm, rowsB_hbm,
                 binnedA, binnedB, slabs, stA, stB, idxA, idxB, idxload,
                 tslabv, offsA, offsB, cursor, semW, semU):
    wid = lax.axis_index("s") * NC + lax.axis_index("c")
    r = wid
    base_k = r * CPT
    ncr = jnp.maximum(0, jnp.minimum(CPT, NCHUNK - base_k))
    lo = base_k * CW
    is_tail = r == TAIL_TILE
    jt = NCHUNK - TAIL_TILE * CPT
    span = ncr * CW + jnp.where(is_tail, 128, 0)

    lane = lax.iota(jnp.int32, L)

    def splat(ref, i):
        return plsc.load_gather(ref, [jnp.full((L,), i, jnp.int32)])

    def bin_build(idx_hbm, binned, offs):
        for q in range(NOFF // L):
            cursor[pl.ds(q * L, L)] = jnp.zeros((L,), jnp.int32)

        def count_round(rd, carry):
            pltpu.sync_copy(idx_hbm.at[pl.ds(rd * 2048, 2048)], idxload)

            def vreg(i, carry):
                v = idxload[pl.ds(i * L, L)]
                dv = v - lo
                m0 = (dv >= 0) & (dv < span)

                def any_hit(state):
                    m, _ = state
                    return jnp.any(m)

                def hit(state):
                    m, carry = state
                    t = plsc.all_reduce_ffs(m)
                    ci = dv.at[t].get(mode="promise_in_bounds") >> 7
                    c_old = plsc.load_gather(cursor, [ci])
                    plsc.store_scatter(cursor, [ci], c_old + 1)
                    return m & (lane != t), carry

                _, carry = lax.while_loop(any_hit, hit, (m0, carry))
                return carry

            return lax.fori_loop(0, 2048 // L, vreg, carry)

        lax.fori_loop(0, BATCH // 2048, count_round, 0)

        def pref(q, carryv):
            c = cursor[pl.ds(q * L, L)]
            incl = c
            for k in (1, 2, 4, 8):
                sh = incl.at[jnp.maximum(lane - k, 0)].get(
                    mode="promise_in_bounds")
                incl = incl + jnp.where(lane >= k, sh, 0)
            excl = incl - c + carryv
            offs[pl.ds(q * L, L)] = excl
            cursor[pl.ds(q * L, L)] = excl
            return incl.at[jnp.full((L,), L - 1, jnp.int32)].get(
                mode="promise_in_bounds") + carryv

        lax.fori_loop(0, NOFF // L, pref, jnp.zeros((L,), jnp.int32))

        def place_round(rd, carry):
            pltpu.sync_copy(idx_hbm.at[pl.ds(rd * 2048, 2048)], idxload)

            def vreg(i, carry):
                v = idxload[pl.ds(i * L, L)]
                dv = v - lo
                m0 = (dv >= 0) & (dv < span)
                b = rd * 2048 + i * L + lane
                p = (dv << 14) | b

                def any_hit(state):
                    m, _ = state
                    return jnp.any(m)

                def hit(state):
                    m, carry = state
                    t = plsc.all_reduce_ffs(m)
                    pt = p.at[t].get(mode="promise_in_bounds")
                    ci = pt >> 21
                    pos = plsc.load_gather(cursor, [ci])
                    posl = jnp.where(lane == 0, pos, BATCH + lane)
                    plsc.store_scatter(binned, [posl], pt)
                    plsc.store_scatter(cursor, [ci], pos + 1)
                    return m & (lane != t), carry

                _, carry = lax.while_loop(any_hit, hit, (m0, carry))
                return carry

            return lax.fori_loop(0, 2048 // L, vreg, carry)

        lax.fori_loop(0, BATCH // 2048, place_round, 0)

    def extract(slab_idx_fn, binned, offs, st, idxst, dst_hbm, j, slot):
        start = splat(offs, j)[0]
        end = splat(offs, j + 1)[0]

        def hit(t, slot):
            pt = splat(binned, t)
            lvec = (pt >> 14) & (CW - 1)
            bvec = pt & 16383
            for g in range(DIM // L):
                dvec = lane + g * L
                vec = plsc.load_gather(*slab_idx_fn(dvec, lvec))
                st[slot, pl.ds(g * L, L)] = vec
            qs = slot >> 4
            ws = slot & 15
            cur = idxst[pl.ds(qs * L, L)]
            idxst[pl.ds(qs * L, L)] = jnp.where(lane == ws, bvec, cur)

            def flush():
                pltpu.sync_copy(st, dst_hbm.at[idxst])
                for q in range(SCAP // L):
                    idxst[pl.ds(q * L, L)] = jnp.full((L,), DUMMY,
                                                      jnp.int32)

            slot = slot + 1
            pl.when(slot == SCAP)(flush)
            return jnp.where(slot == SCAP, 0, slot)

        return lax.fori_loop(start, end, hit, slot)

    for q in range(SCAP // L):
        idxA[pl.ds(q * L, L)] = jnp.full((L,), DUMMY, jnp.int32)
        idxB[pl.ds(q * L, L)] = jnp.full((L,), DUMMY, jnp.int32)

    def issue(j, par):
        col = (base_k + j) * CW
        pltpu.make_async_copy(
            wt_hbm.at[:, pl.ds(col, CW)], slabs.at[2 * par], semW).start()
        pltpu.make_async_copy(
            ut_hbm.at[:, pl.ds(col, CW)], slabs.at[2 * par + 1],
            semU).start()

    def wait(par):
        pltpu.make_async_copy(
            wt_hbm.at[:, pl.ds(0, CW)], slabs.at[2 * par], semW).wait()
        pltpu.make_async_copy(
            ut_hbm.at[:, pl.ds(0, CW)], slabs.at[2 * par + 1], semU).wait()

    for jp in range(NBUF - 1):
        pl.when(jp < ncr)(functools.partial(issue, jp, jp))

    bin_build(c_hbm, binnedA, offsA)
    bin_build(o_hbm, binnedB, offsB)

    def chunk(j, carry):
        slotA, slotB = carry
        par = jnp.remainder(j, NBUF)
        wait(par)
        pl.when(j + NBUF - 1 < ncr)(
            lambda: issue(j + NBUF - 1, jnp.remainder(j + NBUF - 1, NBUF)))

        def slabW(dvec, lvec):
            return (slabs, [jnp.full((L,), 2 * par, jnp.int32), dvec, lvec])

        def slabU(dvec, lvec):
            return (slabs, [jnp.full((L,), 2 * par + 1, jnp.int32), dvec,
                            lvec])

        return slotA, slotB

    slotA, slotB = lax.fori_loop(0, ncr, chunk, (0, 0))

    def tail():
        def tslab_idx(dvec, lvec):
            return (tslabv, [dvec, lvec])

        pltpu.sync_copy(stA, rowsA_hbm.at[idxA])
        pltpu.sync_copy(stB, rowsB_hbm.at[idxB])
        for q in range(SCAP // L):
            idxA[pl.ds(q * L, L)] = jnp.full((L,), DUMMY, jnp.int32)
            idxB[pl.ds(q * L, L)] = jnp.full((L,), DUMMY, jnp.int32)

        pltpu.sync_copy(wt_hbm.at[:, pl.ds(TAIL_COL, 64)], tslabv)
        extract(tslab_idx, binnedA, offsA, stB, idxB, rowsA_hbm, jt, 0)
        pltpu.sync_copy(stB, rowsA_hbm.at[idxB])
        for q in range(SCAP // L):
            idxB[pl.ds(q * L, L)] = jnp.full((L,), DUMMY, jnp.int32)
        pltpu.sync_copy(ut_hbm.at[:, pl.ds(TAIL_COL, 64)], tslabv)
        extract(tslab_idx, binnedB, offsB, stB, idxB, rowsB_hbm, jt, 0)
        pltpu.sync_copy(stB, rowsB_hbm.at[idxB])

    def no_tail():
        pltpu.sync_copy(stA, rowsA_hbm.at[idxA])
        pltpu.sync_copy(stB, rowsB_hbm.at[idxB])

    pl.when(is_tail)(tail)
    pl.when(jnp.logical_not(is_tail))(no_tail)


def _phase2_body(rowsA_hbm, rowsB_hbm, out_hbm, bufs, outv, sem):
    wid = lax.axis_index("s") * NC + lax.axis_index("c")
    base = wid * (BATCH // NW)
    lane = lax.iota(jnp.int32, L)
    perms = [lane ^ k for k in (8, 4, 2, 1)]
    NB = 128

    def issue(blk, par):
        pltpu.make_async_copy(
            rowsA_hbm.at[pl.ds(base + blk * NB, NB), :], bufs.at[2 * par],
            sem).start()
        pltpu.make_async_copy(
            rowsB_hbm.at[pl.ds(base + blk * NB, NB), :],
            bufs.at[2 * par + 1], sem).start()

    def wait(par):
        pltpu.make_async_copy(
            rowsA_hbm.at[pl.ds(0, NB), :], bufs.at[2 * par], sem).wait()
        pltpu.make_async_copy(
            rowsB_hbm.at[pl.ds(0, NB), :], bufs.at[2 * par + 1], sem).wait()

    issue(0, 0)
    nblk = (BATCH // NW) // NB

    def block(blk, carry):
        par = jnp.remainder(blk, 2)
        wait(par)
        pl.when(blk + 1 < nblk)(lambda: issue(blk + 1, 1 - par))

        def group(g, carry2):
            res = jnp.zeros((L,), jnp.float32)
            for i in range(L):
                rr = g * L + i
                acc = (bufs[2 * par, rr, pl.ds(0, L)]
                       * bufs[2 * par + 1, rr, pl.ds(0, L)])
                for c in range(1, DIM // L):
                    acc = acc + (bufs[2 * par, rr, pl.ds(c * L, L)]
                                 * bufs[2 * par + 1, rr, pl.ds(c * L, L)])
                for p in perms:
                    acc = acc + acc.at[p].get(mode="promise_in_bounds")
                res = jnp.where(lane == i, acc, res)
            outv[pl.ds(blk * NB + g * L, L)] = res
            return carry2

        lax.fori_loop(0, NB // L, group, 0)
        return carry

    lax.fori_loop(0, nblk, block, 0)
    pltpu.sync_copy(outv, out_hbm.at[pl.ds(base, BATCH // NW)])


@jax.jit
def kernel(center, context, W_in, W_out):
    mesh = plsc.VectorSubcoreMesh(core_axis_name="c", subcore_axis_name="s")
    run1 = functools.partial(
        pl.kernel,
        mesh=mesh,
        compiler_params=pltpu.CompilerParams(use_tc_tiling_on_sc=True,
                                             needs_layout_passes=False),
        out_type=(
            jax.ShapeDtypeStruct((NSLOT, ROWS_PAD), jnp.float32),
            jax.ShapeDtypeStruct((NSLOT, ROWS_PAD), jnp.float32),
        ),
        scratch_types=[
            pltpu.VMEM((HCAP,), jnp.int32),
            pltpu.VMEM((HCAP,), jnp.int32),
            pltpu.VMEM((2 * NBUF, DIM, CW), jnp.float32),
            pltpu.VMEM((SCAP, ROWS_PAD), jnp.float32),
            pltpu.VMEM((SCAP, ROWS_PAD), jnp.float32),
            pltpu.VMEM((SCAP,), jnp.int32),
            pltpu.VMEM((SCAP,), jnp.int32),
            pltpu.VMEM((2048,), jnp.int32),
            pltpu.VMEM((DIM, 64), jnp.float32),
            pltpu.VMEM((NOFF,), jnp.int32),
            pltpu.VMEM((NOFF,), jnp.int32),
            pltpu.VMEM((NOFF,), jnp.int32),
            pltpu.SemaphoreType.DMA,
            pltpu.SemaphoreType.DMA,
        ],
    )(_phase1_body)
    run2 = functools.partial(
        pl.kernel,
        mesh=mesh,
        compiler_params=pltpu.CompilerParams(use_tc_tiling_on_sc=True),
        out_type=jax.ShapeDtypeStruct((BATCH,), jnp.float32),
        scratch_types=[
            pltpu.VMEM((4, 128, ROWS_PAD), jnp.float32),
            pltpu.VMEM((BATCH // NW,), jnp.float32),
            pltpu.SemaphoreType.DMA,
        ],
    )(_phase2_body)
    rowsA, rowsB = run1(W_in.T, W_out.T, center.astype(jnp.int32),
                        context.astype(jnp.int32))
    return run2(rowsA, rowsB)


# speedup vs baseline: 1.3776x; 1.3776x over previous
import functools

import jax
import jax.numpy as jnp
from jax import lax
from jax.experimental import pallas as pl
from jax.experimental.pallas import tpu as pltpu
from jax.experimental.pallas import tpu_sc as plsc

VOCAB = 1000000
DIM = 64
BATCH = 16384

NC = 2
NS = 16
L = 16
NW = NC * NS

CW = 128
NCHUNK = VOCAB // CW
CPT = 256
TAIL_COL = NCHUNK * CW
TAIL_TILE = 30
OWN_SHIFT = 15

BPT = BATCH // NW
ROWS_PAD = 128
NSLOT = BATCH + 8
DUMMY = BATCH
SCAP = 64
NOFF = 256
NBUF = 3
HCAP = BATCH + 16


def _phase0_body(c_hbm, o_hbm, regA_hbm, regB_hbm, cnts_hbm,
                 idxl, bins, curs, sem):
    wid = lax.axis_index("s") * NC + lax.axis_index("c")
    w = wid
    lane = lax.iota(jnp.int32, L)

    def one_table(src_hbm, reg_hbm, t):
        pltpu.sync_copy(src_hbm.at[pl.ds(w * BPT, BPT)], idxl)
        for q in range(48 // L):
            curs[pl.ds(q * L, L)] = jnp.zeros((L,), jnp.int32)

        def vreg(i, carry):
            v = idxl[pl.ds(i * L, L)]
            b = w * BPT + i * L + lane
            p = ((v & ((1 << OWN_SHIFT) - 1)) << 14) | b
            o = v >> OWN_SHIFT
            for li in range(L):
                lf = jnp.full((L,), li, jnp.int32)
                o_s = o.at[lf].get(mode="promise_in_bounds")
                p_s = p.at[lf].get(mode="promise_in_bounds")
                pos = plsc.load_gather(curs, [o_s])
                posl = jnp.where(lane == 0, pos, 512 + lane)
                plsc.store_scatter(bins, [jnp.where(lane == 0, o_s, 32),
                                          posl], p_s)
                plsc.store_scatter(curs,
                                   [jnp.where(lane == 0, o_s, 32 + lane)],
                                   pos + 1)
            return carry

        lax.fori_loop(0, BPT // L, vreg, 0)

        cps = []
        for o in range(NW):
            cps.append(pltpu.make_async_copy(
                bins.at[o, pl.ds(0, 512)],
                reg_hbm.at[o * NW + w, :], sem))
            cps[-1].start()
        pltpu.sync_copy(curs.at[pl.ds(0, 32)],
                        cnts_hbm.at[pl.ds(t * 1024 + w * 32, 32)])
        for cp in cps:
            cp.wait()

    one_table(c_hbm, regA_hbm, 0)
    one_table(o_hbm, regB_hbm, 1)


def _phase1_body(wt_hbm, ut_hbm, regA_hbm, regB_hbm, cnts_hbm,
                 rowsA_hbm, rowsB_hbm,
                 binnedA, binnedB, mb, cbuf, slabs, stA, stB, idxA, idxB,
                 tslabv, offsA, offsB, cursor, semW, semU):
    wid = lax.axis_index("s") * NC + lax.axis_index("c")
    r = wid
    base_k = r * CPT
    ncr = jnp.maximum(0, jnp.minimum(CPT, NCHUNK - base_k))
    is_tail = r == TAIL_TILE
    jt = NCHUNK - TAIL_TILE * CPT

    lane = lax.iota(jnp.int32, L)

    def splat(ref, i):
        return plsc.load_gather(ref, [jnp.full((L,), i, jnp.int32)])

    def bin_build(reg_hbm, t, binned, offs):
        pltpu.sync_copy(reg_hbm.at[pl.ds(r * NW, NW), :], mb)
        pltpu.sync_copy(cnts_hbm.at[pl.ds(t * 1024, 1024)], cbuf)
        cw_lo = plsc.load_gather(cbuf, [r + 32 * lane])
        cw_hi = plsc.load_gather(cbuf, [r + 32 * (L + lane)])

        for q in range(272 // L):
            cursor[pl.ds(q * L, L)] = jnp.zeros((L,), jnp.int32)

        ones = jnp.full((L,), 1, jnp.int32)

        def scan(w, handler):
            cv = cw_lo if w < L else cw_hi
            cw = cv.at[jnp.full((L,), w % L, jnp.int32)].get(
                mode="promise_in_bounds")
            cw0 = cw[0]
            nv = (cw0 + L - 1) >> 4

            def vreg(i, carry):
                p = mb[w, pl.ds(i * L, L)]
                m0 = (i * L + lane) < cw0
                return handler(p, m0, carry)

            return lax.fori_loop(0, nv, vreg, 0)

        def count_h(p, m0, carry):
            ci = p >> 21
            pos = jnp.where(m0, ci, 256 + lane)
            plsc.addupdate_scatter(cursor, [pos], ones)
            return carry

        for w in range(NW):
            scan(w, count_h)

        def pref(q, carryv):
            c = cursor[pl.ds(q * L, L)]
            incl = c
            for k in (1, 2, 4, 8):
                sh = incl.at[jnp.maximum(lane - k, 0)].get(
                    mode="promise_in_bounds")
                incl = incl + jnp.where(lane >= k, sh, 0)
            excl = incl - c + carryv
            offs[pl.ds(q * L, L)] = excl
            cursor[pl.ds(q * L, L)] = excl
            return incl.at[jnp.full((L,), L - 1, jnp.int32)].get(
                mode="promise_in_bounds") + carryv

        lax.fori_loop(0, NOFF // L, pref, jnp.zeros((L,), jnp.int32))

        def place_h(p, m0, carry):
            def any_hit(state):
                m, _ = state
                return jnp.any(m)

            def hit(state):
                m, carry = state
                tt = plsc.all_reduce_ffs(m)
                pt = p.at[tt].get(mode="promise_in_bounds")
                ci = pt >> 21
                pos = plsc.load_gather(cursor, [ci])
                posl = jnp.where(lane == 0, pos, BATCH + lane)
                plsc.store_scatter(binned, [posl], pt)
                plsc.store_scatter(cursor, [ci], pos + 1)
                return m & (lane != tt), carry

            _, carry = lax.while_loop(any_hit, hit, (m0, carry))
            return carry

        for w in range(NW):
            scan(w, place_h)

    def extract(slab_idx_fn, binned, offs, st, idxst, dst_hbm, j, slot):
        start = splat(offs, j)[0]
        end = splat(offs, j + 1)[0]

        def hit(t, slot):
            pt = splat(binned, t)
            lvec = (pt >> 14) & (CW - 1)
            bvec = pt & 16383
            for g in range(DIM // L):
                dvec = lane + g * L
                vec = plsc.load_gather(*slab_idx_fn(dvec, lvec))
                st[slot, pl.ds(g * L, L)] = vec
            qs = slot >> 4
            ws = slot & 15
            cur = idxst[pl.ds(qs * L, L)]
            idxst[pl.ds(qs * L, L)] = jnp.where(lane == ws, bvec, cur)

            def flush():
                pltpu.sync_copy(st, dst_hbm.at[idxst])
                for q in range(SCAP // L):
                    idxst[pl.ds(q * L, L)] = jnp.full((L,), DUMMY,
                                                      jnp.int32)

            slot = slot + 1
            pl.when(slot == SCAP)(flush)
            return jnp.where(slot == SCAP, 0, slot)

        return lax.fori_loop(start, end, hit, slot)

    for q in range(SCAP // L):
        idxA[pl.ds(q * L, L)] = jnp.full((L,), DUMMY, jnp.int32)
        idxB[pl.ds(q * L, L)] = jnp.full((L,), DUMMY, jnp.int32)

    def issue(j, par):
        col = (base_k + j) * CW
        pltpu.make_async_copy(
            wt_hbm.at[:, pl.ds(col, CW)], slabs.at[2 * par], semW).start()
        pltpu.make_async_copy(
            ut_hbm.at[:, pl.ds(col, CW)], slabs.at[2 * par + 1],
            semU).start()

    def wait(par):
        pltpu.make_async_copy(
            wt_hbm.at[:, pl.ds(0, CW)], slabs.at[2 * par], semW).wait()
        pltpu.make_async_copy(
            ut_hbm.at[:, pl.ds(0, CW)], slabs.at[2 * par + 1], semU).wait()

    for jp in range(NBUF - 1):
        pl.when(jp < ncr)(functools.partial(issue, jp, jp))

    bin_build(regA_hbm, 0, binnedA, offsA)
    bin_build(regB_hbm, 1, binnedB, offsB)

    def chunk(j, carry):
        slotA, slotB = carry
        par = jnp.remainder(j, NBUF)
        wait(par)
        pl.when(j + NBUF - 1 < ncr)(
            lambda: issue(j + NBUF - 1, jnp.remainder(j + NBUF - 1, NBUF)))

        def slabW(dvec, lvec):
            return (slabs, [jnp.full((L,), 2 * par, jnp.int32), dvec, lvec])

        def slabU(dvec, lvec):
            return (slabs, [jnp.full((L,), 2 * par + 1, jnp.int32), dvec,
                            lvec])

        slotA = extract(slabW, binnedA, offsA, stA, idxA, rowsA_hbm, j,
                        slotA)
        slotB = extract(slabU, binnedB, offsB, stB, idxB, rowsB_hbm, j,
                        slotB)
        return slotA, slotB

    slotA, slotB = lax.fori_loop(0, ncr, chunk, (0, 0))

    def tail():
        def tslab_idx(dvec, lvec):
            return (tslabv, [dvec, lvec])

        pltpu.sync_copy(stA, rowsA_hbm.at[idxA])
        pltpu.sync_copy(stB, rowsB_hbm.at[idxB])
        for q in range(SCAP // L):
            idxA[pl.ds(q * L, L)] = jnp.full((L,), DUMMY, jnp.int32)
            idxB[pl.ds(q * L, L)] = jnp.full((L,), DUMMY, jnp.int32)

        pltpu.sync_copy(wt_hbm.at[:, pl.ds(TAIL_COL, 64)], tslabv)
        extract(tslab_idx, binnedA, offsA, stB, idxB, rowsA_hbm, jt, 0)
        pltpu.sync_copy(stB, rowsA_hbm.at[idxB])
        for q in range(SCAP // L):
            idxB[pl.ds(q * L, L)] = jnp.full((L,), DUMMY, jnp.int32)
        pltpu.sync_copy(ut_hbm.at[:, pl.ds(TAIL_COL, 64)], tslabv)
        extract(tslab_idx, binnedB, offsB, stB, idxB, rowsB_hbm, jt, 0)
        pltpu.sync_copy(stB, rowsB_hbm.at[idxB])

    def no_tail():
        pltpu.sync_copy(stA, rowsA_hbm.at[idxA])
        pltpu.sync_copy(stB, rowsB_hbm.at[idxB])

    pl.when(is_tail)(tail)
    pl.when(jnp.logical_not(is_tail))(no_tail)


def _phase2_body(rowsA_hbm, rowsB_hbm, out_hbm, bufs, outv, sem):
    wid = lax.axis_index("s") * NC + lax.axis_index("c")
    base = wid * (BATCH // NW)
    lane = lax.iota(jnp.int32, L)
    perms = [lane ^ k for k in (8, 4, 2, 1)]
    NB = 128

    def issue(blk, par):
        pltpu.make_async_copy(
            rowsA_hbm.at[pl.ds(base + blk * NB, NB), :], bufs.at[2 * par],
            sem).start()
        pltpu.make_async_copy(
            rowsB_hbm.at[pl.ds(base + blk * NB, NB), :],
            bufs.at[2 * par + 1], sem).start()

    def wait(par):
        pltpu.make_async_copy(
            rowsA_hbm.at[pl.ds(0, NB), :], bufs.at[2 * par], sem).wait()
        pltpu.make_async_copy(
            rowsB_hbm.at[pl.ds(0, NB), :], bufs.at[2 * par + 1], sem).wait()

    issue(0, 0)
    nblk = (BATCH // NW) // NB

    def block(blk, carry):
        par = jnp.remainder(blk, 2)
        wait(par)
        pl.when(blk + 1 < nblk)(lambda: issue(blk + 1, 1 - par))

        def group(g, carry2):
            res = jnp.zeros((L,), jnp.float32)
            for i in range(L):
                rr = g * L + i
                acc = (bufs[2 * par, rr, pl.ds(0, L)]
                       * bufs[2 * par + 1, rr, pl.ds(0, L)])
                for c in range(1, DIM // L):
                    acc = acc + (bufs[2 * par, rr, pl.ds(c * L, L)]
                                 * bufs[2 * par + 1, rr, pl.ds(c * L, L)])
                for p in perms:
                    acc = acc + acc.at[p].get(mode="promise_in_bounds")
                res = jnp.where(lane == i, acc, res)
            outv[pl.ds(blk * NB + g * L, L)] = res
            return carry2

        lax.fori_loop(0, NB // L, group, 0)
        return carry

    lax.fori_loop(0, nblk, block, 0)
    pltpu.sync_copy(outv, out_hbm.at[pl.ds(base, BATCH // NW)])


@jax.jit
def kernel(center, context, W_in, W_out):
    mesh = plsc.VectorSubcoreMesh(core_axis_name="c", subcore_axis_name="s")
    cp = pltpu.CompilerParams(use_tc_tiling_on_sc=True,
                              needs_layout_passes=False)
    run0 = functools.partial(
        pl.kernel,
        mesh=mesh,
        compiler_params=cp,
        out_type=(
            jax.ShapeDtypeStruct((NW * NW, 512), jnp.int32),
            jax.ShapeDtypeStruct((NW * NW, 512), jnp.int32),
            jax.ShapeDtypeStruct((2048,), jnp.int32),
        ),
        scratch_types=[
            pltpu.VMEM((BPT,), jnp.int32),
            pltpu.VMEM((33, 528), jnp.int32),
            pltpu.VMEM((48,), jnp.int32),
            pltpu.SemaphoreType.DMA,
        ],
    )(_phase0_body)
    run1 = functools.partial(
        pl.kernel,
        mesh=mesh,
        compiler_params=cp,
        out_type=(
            jax.ShapeDtypeStruct((NSLOT, ROWS_PAD), jnp.float32),
            jax.ShapeDtypeStruct((NSLOT, ROWS_PAD), jnp.float32),
        ),
        scratch_types=[
            pltpu.VMEM((HCAP,), jnp.int32),
            pltpu.VMEM((HCAP,), jnp.int32),
            pltpu.VMEM((NW, 512), jnp.int32),
            pltpu.VMEM((1024,), jnp.int32),
            pltpu.VMEM((2 * NBUF, DIM, CW), jnp.float32),
            pltpu.VMEM((SCAP, ROWS_PAD), jnp.float32),
            pltpu.VMEM((SCAP, ROWS_PAD), jnp.float32),
            pltpu.VMEM((SCAP,), jnp.int32),
            pltpu.VMEM((SCAP,), jnp.int32),
            pltpu.VMEM((DIM, 64), jnp.float32),
            pltpu.VMEM((NOFF,), jnp.int32),
            pltpu.VMEM((NOFF,), jnp.int32),
            pltpu.VMEM((272,), jnp.int32),
            pltpu.SemaphoreType.DMA,
            pltpu.SemaphoreType.DMA,
        ],
    )(_phase1_body)
    run2 = functools.partial(
        pl.kernel,
        mesh=mesh,
        compiler_params=pltpu.CompilerParams(use_tc_tiling_on_sc=True),
        out_type=jax.ShapeDtypeStruct((BATCH,), jnp.float32),
        scratch_types=[
            pltpu.VMEM((4, 128, ROWS_PAD), jnp.float32),
            pltpu.VMEM((BATCH // NW,), jnp.float32),
            pltpu.SemaphoreType.DMA,
        ],
    )(_phase2_body)
    regA, regB, cnts = run0(center.astype(jnp.int32),
                            context.astype(jnp.int32))
    rowsA, rowsB = run1(W_in.T, W_out.T, regA, regB, cnts)
    return run2(rowsA, rowsB)
